# baseline (speedup 1.0000x reference)
import functools
import jax
import jax.numpy as jnp
from jax import lax
from jax.experimental import pallas as pl
from jax.experimental.pallas import tpu as pltpu
from jax.experimental.pallas import tpu_sc as plsc

UNITS = 512
F = 512
BATCH = 256
DEPTH = 3
BLK = 4096
AW = 640
_HI = jax.lax.Precision.HIGHEST


def _f32(a, b):
    return jnp.dot(a, b, preferred_element_type=jnp.float32)


def _split(x):
    hi = x.astype(jnp.bfloat16)
    lo = (x - hi.astype(jnp.float32)).astype(jnp.bfloat16)
    return hi, lo


def _prep_body(node_ref, W_ref, b_ref, w2h_ref, w2l_ref, ba_ref,
               wn_ref, sn_ref):
    x = node_ref[...]
    wn = _f32(x, W_ref[...]) + b_ref[...]
    wn_ref[...] = wn.astype(jnp.bfloat16)
    xh, xl = _split(x)
    sn_ref[...] = (_f32(xh, w2h_ref[...]) + _f32(xh, w2l_ref[...])
                   + _f32(xl, w2h_ref[...])) + ba_ref[...]


NRG = 16
SUB = 128


def _make_h0_sc(rows_per_rg):
    nch = rows_per_rg // SUB
    CH = 256

    @functools.partial(
        pl.kernel,
        mesh=plsc.VectorSubcoreMesh(core_axis_name="c", subcore_axis_name="s"),
        out_type=jax.ShapeDtypeStruct((2, NRG, BATCH, CH), jnp.float32),
        scratch_types=[
            pltpu.VMEM((rows_per_rg,), jnp.int32),
            pltpu.VMEM((SUB, CH), jnp.float32),
            pltpu.VMEM((BATCH, CH), jnp.float32),
        ],
    )
    def _h0_sc(node_hbm, idx_hbm, zero_hbm, out_hbm, bi_v, rows_v, acc_v):
        c = lax.axis_index("c")
        s = lax.axis_index("s")
        pltpu.sync_copy(zero_hbm, acc_v)
        pltpu.sync_copy(idx_hbm.at[s], bi_v)

        def chunk_body(j, carry):
            pltpu.sync_copy(
                node_hbm.at[pl.ds(s * rows_per_rg + j * SUB, SUB),
                            pl.ds(c * CH, CH)], rows_v)

            def grp_body(r16, carry2):
                seg16 = bi_v[pl.ds(j * SUB + r16 * 16, 16)]
                for l in range(16):
                    g = seg16[l]
                    r = r16 * 16 + l
                    for k in range(CH // 16):
                        x = rows_v[r, pl.ds(k * 16, 16)]
                        cur = acc_v[g, pl.ds(k * 16, 16)]
                        acc_v[g, pl.ds(k * 16, 16)] = cur + x
                return carry2

            lax.fori_loop(0, SUB // 16, grp_body, 0)
            return carry

        lax.fori_loop(0, nch, chunk_body, 0)
        pltpu.sync_copy(acc_v, out_hbm.at[c, s])

    return _h0_sc


def _finalize_body(p_ref, wa1_ref, h0_ref, shc_ref, acc_scr):
    i = pl.program_id(0)
    nb = pl.num_programs(0)
    part = jnp.concatenate([p_ref[0, 0], p_ref[1, 0]], axis=-1)

    @pl.when(i == 0)
    def _():
        acc_scr[...] = part

    @pl.when(i > 0)
    def _():
        acc_scr[...] += part

    @pl.when(i == nb - 1)
    def _():
        h0 = acc_scr[...]
        h0_ref[...] = h0
        shc_ref[...] = jnp.dot(h0, wa1_ref[...],
                               preferred_element_type=jnp.float32,
                               precision=_HI)


def _loop_body(wn_ref, sn_ref, bi_ref, sh0_ref, h0_ref,
               gk_ref, gr_ref, gb_ref, wa1_ref, hn_ref,
               acc_ref, h_scr, sh_scr):
    d = pl.program_id(0)
    i = pl.program_id(1)
    nb = pl.num_programs(1)

    @pl.when((d == 0) & (i == 0))
    def _():
        h_scr[...] = h0_ref[...]
        sh_scr[...] = sh0_ref[...]

    bi = bi_ref[0]
    mask = jax.lax.broadcasted_iota(jnp.int32, (BATCH, BLK), 0) == bi
    oh = mask.astype(jnp.bfloat16)
    sh = sh_scr[...]
    shh = sh.astype(jnp.bfloat16)
    shl = (sh - shh.astype(jnp.float32)).astype(jnp.bfloat16)
    shg = _f32(shh, oh) + _f32(shl, oh)
    av = shg[0:1] + sn_ref[0]
    av = jnp.where(av > 0, av, 0.2 * av)
    e = jnp.exp(av)
    ow = jnp.where(mask, jnp.broadcast_to(e, (BATCH, BLK)),
                   0.0).astype(jnp.bfloat16)
    part = _f32(ow, wn_ref[...])

    @pl.when(i == 0)
    def _():
        acc_ref[...] = part

    @pl.when(i > 0)
    def _():
        acc_ref[...] += part

    @pl.when(i == nb - 1)
    def _():
        acc = acc_ref[...]
        denom = jnp.maximum(acc[:, UNITS:UNITS + 1], 1e-30)
        cont = acc[:, :UNITS] / denom
        cont = jnp.where(cont > 0, cont, jnp.exp(cont) - 1.0)
        h = h_scr[...]
        mx = _f32(cont, gk_ref[...]) + gb_ref[0:1]
        mi = _f32(h, gr_ref[...]) + gb_ref[1:2]
        xz, xr, xg = mx[:, :UNITS], mx[:, UNITS:2 * UNITS], mx[:, 2 * UNITS:]
        rz, rr, rg = mi[:, :UNITS], mi[:, UNITS:2 * UNITS], mi[:, 2 * UNITS:]
        z = jax.nn.sigmoid(xz + rz)
        r = jax.nn.sigmoid(xr + rr)
        hh = jnp.tanh(xg + r * rg)
        hn = z * h + (1.0 - z) * hh
        h_scr[...] = hn
        shc = jnp.dot(hn, wa1_ref[...], preferred_element_type=jnp.float32,
                      precision=_HI)
        sh_scr[...] = jnp.broadcast_to(shc[:, 0].reshape(1, BATCH), (8, BATCH))

        @pl.when(d == DEPTH - 1)
        def _():
            hn_ref[...] = hn


def kernel(ref, node, batch_index, W_lin, b_lin, W_alpha, b_alpha,
           gru_kernel, gru_rec, gru_bias):
    N = node.shape[0]
    NB = -(-N // BLK)
    NPAD = NB * BLK
    nodep = jnp.pad(node, ((0, NPAD - N), (0, 0)))
    bip = jnp.pad(batch_index.astype(jnp.int32), (0, NPAD - N),
                  constant_values=BATCH)
    bi3 = bip.reshape(NB, 1, BLK)
    bip0 = jnp.pad(batch_index.astype(jnp.int32), (0, NPAD - N))

    W_aug = jnp.zeros((F, AW), jnp.float32).at[:, :UNITS].set(W_lin)
    b_aug = (jnp.zeros((AW,), jnp.float32)
             .at[:UNITS].set(b_lin)
             .at[UNITS].set(1.0)).reshape(1, AW)
    wa2 = jnp.zeros((F, 128), jnp.float32).at[:, 0].set(W_alpha[UNITS:, 0])
    w2h = wa2.astype(jnp.bfloat16)
    w2l = (wa2 - w2h.astype(jnp.float32)).astype(jnp.bfloat16)
    ba_row = jnp.broadcast_to(b_alpha.reshape(1, 1), (1, 128))
    wa1 = jnp.zeros((UNITS, 128), jnp.float32).at[:, 0].set(W_alpha[:UNITS, 0])
    gbp = jnp.zeros((8, 3 * UNITS), jnp.float32).at[:2].set(gru_bias)

    def full(shape):
        nzero = len(shape)
        return pl.BlockSpec(shape, lambda *args, _n=nzero: (0,) * _n)

    rows_per_rg = NPAD // NRG
    zero_half = jnp.zeros((BATCH, 256), jnp.float32)
    idx_sc = bip0.reshape(NRG, rows_per_rg)
    h0p = _make_h0_sc(rows_per_rg)(nodep, idx_sc, zero_half)

    wn_aug, sn_col = pl.pallas_call(
        _prep_body,
        grid=(NB,),
        in_specs=[
            pl.BlockSpec((BLK, F), lambda i: (i, 0)),
            full((F, AW)),
            full((1, AW)),
            full((F, 128)),
            full((F, 128)),
            full((1, 128)),
        ],
        out_specs=[
            pl.BlockSpec((BLK, AW), lambda i: (i, 0)),
            pl.BlockSpec((BLK, 128), lambda i: (i, 0)),
        ],
        out_shape=[
            jax.ShapeDtypeStruct((NPAD, AW), jnp.bfloat16),
            jax.ShapeDtypeStruct((NPAD, 128), jnp.float32),
        ],
    )(nodep, W_aug, b_aug, w2h, w2l, ba_row)

    h0, sh_col = pl.pallas_call(
        _finalize_body,
        grid=(NRG,),
        in_specs=[
            pl.BlockSpec((2, 1, BATCH, 256), lambda i: (0, i, 0, 0)),
            full((UNITS, 128)),
        ],
        out_specs=[
            full((BATCH, F)),
            full((BATCH, 128)),
        ],
        out_shape=[
            jax.ShapeDtypeStruct((BATCH, F), jnp.float32),
            jax.ShapeDtypeStruct((BATCH, 128), jnp.float32),
        ],
        scratch_shapes=[pltpu.VMEM((BATCH, F), jnp.float32)],
    )(h0p, wa1)

    sn_row = sn_col[:, 0].reshape(NB, 1, BLK)
    sh08 = jnp.broadcast_to(sh_col[:, 0].reshape(1, BATCH), (8, BATCH))

    h = pl.pallas_call(
        _loop_body,
        grid=(DEPTH, NB),
        in_specs=[
            pl.BlockSpec((BLK, AW), lambda d, i: (i, 0)),
            pl.BlockSpec((1, 1, BLK), lambda d, i: (i, 0, 0)),
            pl.BlockSpec((1, 1, BLK), lambda d, i: (i, 0, 0)),
            full((8, BATCH)),
            full((BATCH, UNITS)),
            full((UNITS, 3 * UNITS)),
            full((UNITS, 3 * UNITS)),
            full((8, 3 * UNITS)),
            full((UNITS, 128)),
        ],
        out_specs=full((BATCH, UNITS)),
        out_shape=jax.ShapeDtypeStruct((BATCH, UNITS), jnp.float32),
        scratch_shapes=[
            pltpu.VMEM((BATCH, AW), jnp.float32),
            pltpu.VMEM((BATCH, UNITS), jnp.float32),
            pltpu.VMEM((8, BATCH), jnp.float32),
        ],
    )(wn_aug, sn_row, bi3, sh08, h0, gru_kernel, gru_rec, gbp, wa1)
    return h


# speedup vs baseline: 1.3930x; 1.3930x over previous
import functools
import jax
import jax.numpy as jnp
from jax import lax
from jax.experimental import pallas as pl
from jax.experimental.pallas import tpu as pltpu
from jax.experimental.pallas import tpu_sc as plsc

UNITS = 512
F = 512
BATCH = 256
DEPTH = 3
BLK = 4096
AW = 640
_HI = jax.lax.Precision.HIGHEST


def _f32(a, b):
    return jnp.dot(a, b, preferred_element_type=jnp.float32)


def _split(x):
    hi = x.astype(jnp.bfloat16)
    lo = (x - hi.astype(jnp.float32)).astype(jnp.bfloat16)
    return hi, lo


def _prep_body(node_ref, W_ref, b_ref, w2h_ref, w2l_ref, ba_ref,
               wn_ref, sn_ref):
    x = node_ref[...]
    wn = _f32(x, W_ref[...]) + b_ref[...]
    wn_ref[...] = wn.astype(jnp.bfloat16)
    xh, xl = _split(x)
    sn_ref[...] = (_f32(xh, w2h_ref[...]) + _f32(xh, w2l_ref[...])
                   + _f32(xl, w2h_ref[...])) + ba_ref[...]


NRG = 16
SUB = 128


def _make_h0_sc(rows_per_rg):
    nch = rows_per_rg // SUB
    CH = 256

    @functools.partial(
        pl.kernel,
        mesh=plsc.VectorSubcoreMesh(core_axis_name="c", subcore_axis_name="s"),
        out_type=jax.ShapeDtypeStruct((2, NRG, BATCH, CH), jnp.float32),
        scratch_types=[
            pltpu.VMEM((rows_per_rg,), jnp.int32),
            pltpu.VMEM((SUB, CH), jnp.float32),
            pltpu.VMEM((BATCH, CH), jnp.float32),
        ],
    )
    def _h0_sc(node_hbm, idx_hbm, zero_hbm, out_hbm, bi_v, rows_v, acc_v):
        c = lax.axis_index("c")
        s = lax.axis_index("s")
        pltpu.sync_copy(zero_hbm, acc_v)
        pltpu.sync_copy(idx_hbm.at[s], bi_v)

        def chunk_body(j, carry):
            pltpu.sync_copy(
                node_hbm.at[pl.ds(s * rows_per_rg + j * SUB, SUB),
                            pl.ds(c * CH, CH)], rows_v)

            def grp_body(r16, carry2):
                seg16 = bi_v[pl.ds(j * SUB + r16 * 16, 16)]
                g0 = seg16[0]
                same = g0 == seg16[15]

                @pl.when(same)
                def _():
                    for k in range(CH // 16):
                        t = rows_v[r16 * 16, pl.ds(k * 16, 16)]
                        for l in range(1, 16):
                            t = t + rows_v[r16 * 16 + l, pl.ds(k * 16, 16)]
                        cur = acc_v[g0, pl.ds(k * 16, 16)]
                        acc_v[g0, pl.ds(k * 16, 16)] = cur + t

                @pl.when(jnp.logical_not(same))
                def _():
                    for l in range(16):
                        g = seg16[l]
                        r = r16 * 16 + l
                        for k in range(CH // 16):
                            x = rows_v[r, pl.ds(k * 16, 16)]
                            cur = acc_v[g, pl.ds(k * 16, 16)]
                            acc_v[g, pl.ds(k * 16, 16)] = cur + x
                return carry2

            lax.fori_loop(0, SUB // 16, grp_body, 0)
            return carry

        lax.fori_loop(0, nch, chunk_body, 0)
        pltpu.sync_copy(acc_v, out_hbm.at[c, s])

    return _h0_sc


def _finalize_body(p_ref, wa1_ref, h0_ref, shc_ref, acc_scr):
    i = pl.program_id(0)
    nb = pl.num_programs(0)
    part = jnp.concatenate([p_ref[0, 0], p_ref[1, 0]], axis=-1)

    @pl.when(i == 0)
    def _():
        acc_scr[...] = part

    @pl.when(i > 0)
    def _():
        acc_scr[...] += part

    @pl.when(i == nb - 1)
    def _():
        h0 = acc_scr[...]
        h0_ref[...] = h0
        shc_ref[...] = jnp.dot(h0, wa1_ref[...],
                               preferred_element_type=jnp.float32,
                               precision=_HI)


def _loop_body(wn_ref, sn_ref, bi_ref, sh0_ref, h0_ref,
               gk_ref, gr_ref, gb_ref, wa1_ref, hn_ref,
               acc_ref, h_scr, sh_scr):
    d = pl.program_id(0)
    i = pl.program_id(1)
    nb = pl.num_programs(1)

    @pl.when((d == 0) & (i == 0))
    def _():
        h_scr[...] = h0_ref[...]
        sh_scr[...] = sh0_ref[...]

    bi = bi_ref[0]
    mask = jax.lax.broadcasted_iota(jnp.int32, (BATCH, BLK), 0) == bi
    oh = mask.astype(jnp.bfloat16)
    sh = sh_scr[...]
    shh = sh.astype(jnp.bfloat16)
    shl = (sh - shh.astype(jnp.float32)).astype(jnp.bfloat16)
    shg = _f32(shh, oh) + _f32(shl, oh)
    av = shg[0:1] + sn_ref[0]
    av = jnp.where(av > 0, av, 0.2 * av)
    e = jnp.exp(av)
    ow = jnp.where(mask, jnp.broadcast_to(e, (BATCH, BLK)),
                   0.0).astype(jnp.bfloat16)
    part = _f32(ow, wn_ref[...])

    @pl.when(i == 0)
    def _():
        acc_ref[...] = part

    @pl.when(i > 0)
    def _():
        acc_ref[...] += part

    @pl.when(i == nb - 1)
    def _():
        acc = acc_ref[...]
        denom = jnp.maximum(acc[:, UNITS:UNITS + 1], 1e-30)
        cont = acc[:, :UNITS] / denom
        cont = jnp.where(cont > 0, cont, jnp.exp(cont) - 1.0)
        h = h_scr[...]
        mx = _f32(cont, gk_ref[...]) + gb_ref[0:1]
        mi = _f32(h, gr_ref[...]) + gb_ref[1:2]
        xz, xr, xg = mx[:, :UNITS], mx[:, UNITS:2 * UNITS], mx[:, 2 * UNITS:]
        rz, rr, rg = mi[:, :UNITS], mi[:, UNITS:2 * UNITS], mi[:, 2 * UNITS:]
        z = jax.nn.sigmoid(xz + rz)
        r = jax.nn.sigmoid(xr + rr)
        hh = jnp.tanh(xg + r * rg)
        hn = z * h + (1.0 - z) * hh
        h_scr[...] = hn
        shc = jnp.dot(hn, wa1_ref[...], preferred_element_type=jnp.float32,
                      precision=_HI)
        sh_scr[...] = jnp.broadcast_to(shc[:, 0].reshape(1, BATCH), (8, BATCH))

        @pl.when(d == DEPTH - 1)
        def _():
            hn_ref[...] = hn


def kernel(ref, node, batch_index, W_lin, b_lin, W_alpha, b_alpha,
           gru_kernel, gru_rec, gru_bias):
    N = node.shape[0]
    NB = -(-N // BLK)
    NPAD = NB * BLK
    nodep = jnp.pad(node, ((0, NPAD - N), (0, 0)))
    bip = jnp.pad(batch_index.astype(jnp.int32), (0, NPAD - N),
                  constant_values=BATCH)
    bi3 = bip.reshape(NB, 1, BLK)
    bip0 = jnp.pad(batch_index.astype(jnp.int32), (0, NPAD - N))

    W_aug = jnp.zeros((F, AW), jnp.float32).at[:, :UNITS].set(W_lin)
    b_aug = (jnp.zeros((AW,), jnp.float32)
             .at[:UNITS].set(b_lin)
             .at[UNITS].set(1.0)).reshape(1, AW)
    wa2 = jnp.zeros((F, 128), jnp.float32).at[:, 0].set(W_alpha[UNITS:, 0])
    w2h = wa2.astype(jnp.bfloat16)
    w2l = (wa2 - w2h.astype(jnp.float32)).astype(jnp.bfloat16)
    ba_row = jnp.broadcast_to(b_alpha.reshape(1, 1), (1, 128))
    wa1 = jnp.zeros((UNITS, 128), jnp.float32).at[:, 0].set(W_alpha[:UNITS, 0])
    gbp = jnp.zeros((8, 3 * UNITS), jnp.float32).at[:2].set(gru_bias)

    def full(shape):
        nzero = len(shape)
        return pl.BlockSpec(shape, lambda *args, _n=nzero: (0,) * _n)

    rows_per_rg = NPAD // NRG
    zero_half = jnp.zeros((BATCH, 256), jnp.float32)
    idx_sc = bip0.reshape(NRG, rows_per_rg)
    h0p = _make_h0_sc(rows_per_rg)(nodep, idx_sc, zero_half)

    wn_aug, sn_col = pl.pallas_call(
        _prep_body,
        grid=(NB,),
        in_specs=[
            pl.BlockSpec((BLK, F), lambda i: (i, 0)),
            full((F, AW)),
            full((1, AW)),
            full((F, 128)),
            full((F, 128)),
            full((1, 128)),
        ],
        out_specs=[
            pl.BlockSpec((BLK, AW), lambda i: (i, 0)),
            pl.BlockSpec((BLK, 128), lambda i: (i, 0)),
        ],
        out_shape=[
            jax.ShapeDtypeStruct((NPAD, AW), jnp.bfloat16),
            jax.ShapeDtypeStruct((NPAD, 128), jnp.float32),
        ],
    )(nodep, W_aug, b_aug, w2h, w2l, ba_row)

    h0, sh_col = pl.pallas_call(
        _finalize_body,
        grid=(NRG,),
        in_specs=[
            pl.BlockSpec((2, 1, BATCH, 256), lambda i: (0, i, 0, 0)),
            full((UNITS, 128)),
        ],
        out_specs=[
            full((BATCH, F)),
            full((BATCH, 128)),
        ],
        out_shape=[
            jax.ShapeDtypeStruct((BATCH, F), jnp.float32),
            jax.ShapeDtypeStruct((BATCH, 128), jnp.float32),
        ],
        scratch_shapes=[pltpu.VMEM((BATCH, F), jnp.float32)],
    )(h0p, wa1)

    sn_row = sn_col[:, 0].reshape(NB, 1, BLK)
    sh08 = jnp.broadcast_to(sh_col[:, 0].reshape(1, BATCH), (8, BATCH))

    h = pl.pallas_call(
        _loop_body,
        grid=(DEPTH, NB),
        in_specs=[
            pl.BlockSpec((BLK, AW), lambda d, i: (i, 0)),
            pl.BlockSpec((1, 1, BLK), lambda d, i: (i, 0, 0)),
            pl.BlockSpec((1, 1, BLK), lambda d, i: (i, 0, 0)),
            full((8, BATCH)),
            full((BATCH, UNITS)),
            full((UNITS, 3 * UNITS)),
            full((UNITS, 3 * UNITS)),
            full((8, 3 * UNITS)),
            full((UNITS, 128)),
        ],
        out_specs=full((BATCH, UNITS)),
        out_shape=jax.ShapeDtypeStruct((BATCH, UNITS), jnp.float32),
        scratch_shapes=[
            pltpu.VMEM((BATCH, AW), jnp.float32),
            pltpu.VMEM((BATCH, UNITS), jnp.float32),
            pltpu.VMEM((8, BATCH), jnp.float32),
        ],
    )(wn_aug, sn_row, bi3, sh08, h0, gru_kernel, gru_rec, gbp, wa1)
    return h


# speedup vs baseline: 1.5359x; 1.1026x over previous
import functools
import jax
import jax.numpy as jnp
from jax import lax
from jax.experimental import pallas as pl
from jax.experimental.pallas import tpu as pltpu
from jax.experimental.pallas import tpu_sc as plsc

UNITS = 512
F = 512
BATCH = 256
DEPTH = 3
BLK = 4096
AW = 640
_HI = jax.lax.Precision.HIGHEST


def _f32(a, b):
    return jnp.dot(a, b, preferred_element_type=jnp.float32)


def _split(x):
    hi = x.astype(jnp.bfloat16)
    lo = (x - hi.astype(jnp.float32)).astype(jnp.bfloat16)
    return hi, lo


TC_BLK0 = 8


def _prep_body(node_ref, W_ref, b_ref, w2h_ref, w2l_ref, ba_ref, bi_ref,
               wn_ref, sn_ref, h0t_ref, acc_ref):
    i = pl.program_id(0)
    nb = pl.num_programs(0)
    x = node_ref[...]
    wn = _f32(x, W_ref[...]) + b_ref[...]
    wn_ref[...] = wn.astype(jnp.bfloat16)
    xh, xl = _split(x)
    sn_ref[...] = (_f32(xh, w2h_ref[...]) + _f32(xh, w2l_ref[...])
                   + _f32(xl, w2h_ref[...])) + ba_ref[...]

    @pl.when(i >= TC_BLK0)
    def _():
        bi = bi_ref[0]
        oh = (jax.lax.broadcasted_iota(jnp.int32, (BATCH, BLK), 0) == bi
              ).astype(jnp.bfloat16)
        part = _f32(oh, xh) + _f32(oh, xl)

        @pl.when(i == TC_BLK0)
        def _():
            acc_ref[...] = part

        @pl.when(i > TC_BLK0)
        def _():
            acc_ref[...] += part

    @pl.when(i == nb - 1)
    def _():
        h0t_ref[...] = acc_ref[...]


NRG = 16
SUB = 128


def _make_h0_sc(rows_per_rg):
    nch = rows_per_rg // SUB
    CH = 256

    @functools.partial(
        pl.kernel,
        mesh=plsc.VectorSubcoreMesh(core_axis_name="c", subcore_axis_name="s"),
        out_type=jax.ShapeDtypeStruct((2, NRG, BATCH, CH), jnp.float32),
        scratch_types=[
            pltpu.VMEM((rows_per_rg,), jnp.int32),
            pltpu.VMEM((SUB, CH), jnp.float32),
            pltpu.VMEM((BATCH, CH), jnp.float32),
        ],
    )
    def _h0_sc(node_hbm, idx_hbm, zero_hbm, out_hbm, bi_v, rows_v, acc_v):
        c = lax.axis_index("c")
        s = lax.axis_index("s")
        pltpu.sync_copy(zero_hbm, acc_v)
        pltpu.sync_copy(idx_hbm.at[s], bi_v)

        def chunk_body(j, carry):
            pltpu.sync_copy(
                node_hbm.at[pl.ds(s * rows_per_rg + j * SUB, SUB),
                            pl.ds(c * CH, CH)], rows_v)

            def grp_body(r16, carry2):
                seg16 = bi_v[pl.ds(j * SUB + r16 * 16, 16)]
                g0 = seg16[0]
                same = g0 == seg16[15]

                @pl.when(same)
                def _():
                    for k in range(CH // 16):
                        t = rows_v[r16 * 16, pl.ds(k * 16, 16)]
                        for l in range(1, 16):
                            t = t + rows_v[r16 * 16 + l, pl.ds(k * 16, 16)]
                        cur = acc_v[g0, pl.ds(k * 16, 16)]
                        acc_v[g0, pl.ds(k * 16, 16)] = cur + t

                @pl.when(jnp.logical_not(same))
                def _():
                    for l in range(16):
                        g = seg16[l]
                        r = r16 * 16 + l
                        for k in range(CH // 16):
                            x = rows_v[r, pl.ds(k * 16, 16)]
                            cur = acc_v[g, pl.ds(k * 16, 16)]
                            acc_v[g, pl.ds(k * 16, 16)] = cur + x
                return carry2

            lax.fori_loop(0, SUB // 16, grp_body, 0)
            return carry

        lax.fori_loop(0, nch, chunk_body, 0)
        pltpu.sync_copy(acc_v, out_hbm.at[c, s])

    return _h0_sc


def _finalize_body(p_ref, h0t_ref, wa1_ref, h0_ref, shc_ref, acc_scr):
    i = pl.program_id(0)
    nb = pl.num_programs(0)
    part = jnp.concatenate([p_ref[0, 0], p_ref[1, 0]], axis=-1)

    @pl.when(i == 0)
    def _():
        acc_scr[...] = part + h0t_ref[...]

    @pl.when(i > 0)
    def _():
        acc_scr[...] += part

    @pl.when(i == nb - 1)
    def _():
        h0 = acc_scr[...]
        h0_ref[...] = h0
        shc_ref[...] = jnp.dot(h0, wa1_ref[...],
                               preferred_element_type=jnp.float32,
                               precision=_HI)


def _loop_body(wn_ref, sn_ref, bi_ref, sh0_ref, h0_ref,
               gk_ref, gr_ref, gb_ref, wa1_ref, hn_ref,
               acc_ref, h_scr, sh_scr):
    d = pl.program_id(0)
    i = pl.program_id(1)
    nb = pl.num_programs(1)

    @pl.when((d == 0) & (i == 0))
    def _():
        h_scr[...] = h0_ref[...]
        sh_scr[...] = sh0_ref[...]

    bi = bi_ref[0]
    mask = jax.lax.broadcasted_iota(jnp.int32, (BATCH, BLK), 0) == bi
    oh = mask.astype(jnp.bfloat16)
    sh = sh_scr[...]
    shh = sh.astype(jnp.bfloat16)
    shl = (sh - shh.astype(jnp.float32)).astype(jnp.bfloat16)
    shg = _f32(shh, oh) + _f32(shl, oh)
    av = shg[0:1] + sn_ref[0]
    av = jnp.where(av > 0, av, 0.2 * av)
    e = jnp.exp(av)
    ow = jnp.where(mask, jnp.broadcast_to(e, (BATCH, BLK)),
                   0.0).astype(jnp.bfloat16)
    part = _f32(ow, wn_ref[...])

    @pl.when(i == 0)
    def _():
        acc_ref[...] = part

    @pl.when(i > 0)
    def _():
        acc_ref[...] += part

    @pl.when(i == nb - 1)
    def _():
        acc = acc_ref[...]
        denom = jnp.maximum(acc[:, UNITS:UNITS + 1], 1e-30)
        cont = acc[:, :UNITS] / denom
        cont = jnp.where(cont > 0, cont, jnp.exp(cont) - 1.0)
        h = h_scr[...]
        mx = _f32(cont, gk_ref[...]) + gb_ref[0:1]
        mi = _f32(h, gr_ref[...]) + gb_ref[1:2]
        xz, xr, xg = mx[:, :UNITS], mx[:, UNITS:2 * UNITS], mx[:, 2 * UNITS:]
        rz, rr, rg = mi[:, :UNITS], mi[:, UNITS:2 * UNITS], mi[:, 2 * UNITS:]
        z = jax.nn.sigmoid(xz + rz)
        r = jax.nn.sigmoid(xr + rr)
        hh = jnp.tanh(xg + r * rg)
        hn = z * h + (1.0 - z) * hh
        h_scr[...] = hn
        shc = jnp.dot(hn, wa1_ref[...], preferred_element_type=jnp.float32,
                      precision=_HI)
        sh_scr[...] = jnp.broadcast_to(shc[:, 0].reshape(1, BATCH), (8, BATCH))

        @pl.when(d == DEPTH - 1)
        def _():
            hn_ref[...] = hn


def kernel(ref, node, batch_index, W_lin, b_lin, W_alpha, b_alpha,
           gru_kernel, gru_rec, gru_bias):
    N = node.shape[0]
    NB = -(-N // BLK)
    NPAD = NB * BLK
    nodep = jnp.pad(node, ((0, NPAD - N), (0, 0)))
    bip = jnp.pad(batch_index.astype(jnp.int32), (0, NPAD - N),
                  constant_values=BATCH)
    bi3 = bip.reshape(NB, 1, BLK)
    bip0 = jnp.pad(batch_index.astype(jnp.int32), (0, NPAD - N))

    W_aug = jnp.zeros((F, AW), jnp.float32).at[:, :UNITS].set(W_lin)
    b_aug = (jnp.zeros((AW,), jnp.float32)
             .at[:UNITS].set(b_lin)
             .at[UNITS].set(1.0)).reshape(1, AW)
    wa2 = jnp.zeros((F, 128), jnp.float32).at[:, 0].set(W_alpha[UNITS:, 0])
    w2h = wa2.astype(jnp.bfloat16)
    w2l = (wa2 - w2h.astype(jnp.float32)).astype(jnp.bfloat16)
    ba_row = jnp.broadcast_to(b_alpha.reshape(1, 1), (1, 128))
    wa1 = jnp.zeros((UNITS, 128), jnp.float32).at[:, 0].set(W_alpha[:UNITS, 0])
    gbp = jnp.zeros((8, 3 * UNITS), jnp.float32).at[:2].set(gru_bias)

    def full(shape):
        nzero = len(shape)
        return pl.BlockSpec(shape, lambda *args, _n=nzero: (0,) * _n)

    nsc_rows = TC_BLK0 * BLK
    rows_per_rg = nsc_rows // NRG
    zero_half = jnp.zeros((BATCH, 256), jnp.float32)
    idx_sc = bip0[:nsc_rows].reshape(NRG, rows_per_rg)
    h0p = _make_h0_sc(rows_per_rg)(nodep, idx_sc, zero_half)

    wn_aug, sn_col, h0t = pl.pallas_call(
        _prep_body,
        grid=(NB,),
        in_specs=[
            pl.BlockSpec((BLK, F), lambda i: (i, 0)),
            full((F, AW)),
            full((1, AW)),
            full((F, 128)),
            full((F, 128)),
            full((1, 128)),
            pl.BlockSpec((1, 1, BLK), lambda i: (i, 0, 0)),
        ],
        out_specs=[
            pl.BlockSpec((BLK, AW), lambda i: (i, 0)),
            pl.BlockSpec((BLK, 128), lambda i: (i, 0)),
            full((BATCH, F)),
        ],
        out_shape=[
            jax.ShapeDtypeStruct((NPAD, AW), jnp.bfloat16),
            jax.ShapeDtypeStruct((NPAD, 128), jnp.float32),
            jax.ShapeDtypeStruct((BATCH, F), jnp.float32),
        ],
        scratch_shapes=[pltpu.VMEM((BATCH, F), jnp.float32)],
    )(nodep, W_aug, b_aug, w2h, w2l, ba_row, bi3)

    h0, sh_col = pl.pallas_call(
        _finalize_body,
        grid=(NRG,),
        in_specs=[
            pl.BlockSpec((2, 1, BATCH, 256), lambda i: (0, i, 0, 0)),
            full((BATCH, F)),
            full((UNITS, 128)),
        ],
        out_specs=[
            full((BATCH, F)),
            full((BATCH, 128)),
        ],
        out_shape=[
            jax.ShapeDtypeStruct((BATCH, F), jnp.float32),
            jax.ShapeDtypeStruct((BATCH, 128), jnp.float32),
        ],
        scratch_shapes=[pltpu.VMEM((BATCH, F), jnp.float32)],
    )(h0p, h0t, wa1)

    sn_row = sn_col[:, 0].reshape(NB, 1, BLK)
    sh08 = jnp.broadcast_to(sh_col[:, 0].reshape(1, BATCH), (8, BATCH))

    h = pl.pallas_call(
        _loop_body,
        grid=(DEPTH, NB),
        in_specs=[
            pl.BlockSpec((BLK, AW), lambda d, i: (i, 0)),
            pl.BlockSpec((1, 1, BLK), lambda d, i: (i, 0, 0)),
            pl.BlockSpec((1, 1, BLK), lambda d, i: (i, 0, 0)),
            full((8, BATCH)),
            full((BATCH, UNITS)),
            full((UNITS, 3 * UNITS)),
            full((UNITS, 3 * UNITS)),
            full((8, 3 * UNITS)),
            full((UNITS, 128)),
        ],
        out_specs=full((BATCH, UNITS)),
        out_shape=jax.ShapeDtypeStruct((BATCH, UNITS), jnp.float32),
        scratch_shapes=[
            pltpu.VMEM((BATCH, AW), jnp.float32),
            pltpu.VMEM((BATCH, UNITS), jnp.float32),
            pltpu.VMEM((8, BATCH), jnp.float32),
        ],
    )(wn_aug, sn_row, bi3, sh08, h0, gru_kernel, gru_rec, gbp, wa1)
    return h
